# baseline (speedup 1.0000x reference)
import functools

import jax
import jax.numpy as jnp
from jax import lax
from jax.experimental import pallas as pl
from jax.experimental.pallas import tpu as pltpu
from jax.experimental.pallas import tpu_sc as plsc


def _permute_columns_sc(x, permutation):
  R, F = x.shape
  NC, NS, L = 2, 16, 16
  NW = NC * NS
  rows_per_w = R // NW
  CH = 128
  n_chunks = rows_per_w // CH
  n_groups = F // L

  mesh = plsc.VectorSubcoreMesh(core_axis_name="c", subcore_axis_name="s")

  @functools.partial(
      pl.kernel,
      mesh=mesh,
      compiler_params=pltpu.CompilerParams(needs_layout_passes=False),
      out_type=jax.ShapeDtypeStruct((R * F,), x.dtype),
      scratch_types=[
          pltpu.VMEM((F,), jnp.int32),
          pltpu.VMEM((CH * F,), x.dtype),
          pltpu.VMEM((CH * F,), x.dtype),
      ],
  )
  def k(x_hbm, perm_hbm, out_hbm, perm_v, in_v, out_v):
    wid = lax.axis_index("c") * NS + lax.axis_index("s")
    base = wid * rows_per_w
    pltpu.sync_copy(perm_hbm, perm_v)

    def chunk_body(c, carry):
      e0 = (base + c * CH) * F
      pltpu.sync_copy(x_hbm.at[pl.ds(e0, CH * F)], in_v)
      for g in range(n_groups):
        idx = perm_v[pl.ds(g * L, L)]

        @plsc.parallel_loop(0, CH, unroll=4)
        def row_body(i):
          off = i * F
          flat = jnp.full((L,), off, dtype=jnp.int32) + idx
          out_v[pl.ds(off + g * L, L)] = plsc.load_gather(in_v, [flat])

      pltpu.sync_copy(out_v, out_hbm.at[pl.ds(e0, CH * F)])
      return carry

    lax.fori_loop(0, n_chunks, chunk_body, 0)

  return k(x.reshape(R * F), permutation).reshape(R, F)


def kernel(x, permutation):
  return _permute_columns_sc(x, permutation)


# speedup vs baseline: 1.1926x; 1.1926x over previous
import functools

import jax
import jax.numpy as jnp
from jax import lax
from jax.experimental import pallas as pl
from jax.experimental.pallas import tpu as pltpu
from jax.experimental.pallas import tpu_sc as plsc


def _permute_columns_sc(x, permutation):
  R, F = x.shape
  NC, NS, L = 2, 16, 16
  NW = NC * NS
  rows_per_w = R // NW
  CH = 64
  n_chunks = rows_per_w // CH
  n_groups = F // L

  mesh = plsc.VectorSubcoreMesh(core_axis_name="c", subcore_axis_name="s")

  @functools.partial(
      pl.kernel,
      mesh=mesh,
      compiler_params=pltpu.CompilerParams(needs_layout_passes=False),
      out_type=jax.ShapeDtypeStruct((R * F,), x.dtype),
      scratch_types=[
          pltpu.VMEM((F,), jnp.int32),
          [pltpu.VMEM((CH * F,), x.dtype) for _ in range(2)],
          [pltpu.VMEM((CH * F,), x.dtype) for _ in range(2)],
          [pltpu.SemaphoreType.DMA for _ in range(2)],
          [pltpu.SemaphoreType.DMA for _ in range(2)],
      ],
  )
  def k(x_hbm, perm_hbm, out_hbm, perm_v, in_v, out_v, in_sem, out_sem):
    wid = lax.axis_index("c") * NS + lax.axis_index("s")
    base = wid * rows_per_w
    pltpu.sync_copy(perm_hbm, perm_v)

    def in_copy(c, b):
      e0 = (base + c * CH) * F
      return pltpu.make_async_copy(
          x_hbm.at[pl.ds(e0, CH * F)], in_v[b], in_sem[b])

    def out_copy(c, b):
      e0 = (base + c * CH) * F
      return pltpu.make_async_copy(
          out_v[b], out_hbm.at[pl.ds(e0, CH * F)], out_sem[b])

    for b in range(2):
      in_copy(b, b).start()

    def chunk_pair(cc, carry):
      for b in range(2):
        c = cc * 2 + b
        in_copy(c, b).wait()

        @pl.when(c >= 2)
        def _():
          out_copy(c - 2, b).wait()

        for g in range(n_groups):
          idx = perm_v[pl.ds(g * L, L)]

          @plsc.parallel_loop(0, CH, unroll=4)
          def row_body(i):
            off = i * F
            flat = jnp.full((L,), off, dtype=jnp.int32) + idx
            out_v[b][pl.ds(off + g * L, L)] = plsc.load_gather(in_v[b], [flat])

        out_copy(c, b).start()

        @pl.when(c + 2 < n_chunks)
        def _():
          in_copy(c + 2, b).start()

      return carry

    lax.fori_loop(0, n_chunks // 2, chunk_pair, 0)

    for b in range(2):
      out_copy(n_chunks - 2 + b, b).wait()

  return k(x.reshape(R * F), permutation).reshape(R, F)


def kernel(x, permutation):
  return _permute_columns_sc(x, permutation)


# speedup vs baseline: 1.2457x; 1.0445x over previous
import functools

import jax
import jax.numpy as jnp
from jax import lax
from jax.experimental import pallas as pl
from jax.experimental.pallas import tpu as pltpu
from jax.experimental.pallas import tpu_sc as plsc


def _permute_columns_sc(x, permutation):
  R, F = x.shape
  NC, NS, L = 2, 16, 16
  NW = NC * NS
  rows_per_w = R // NW
  CH = 64
  n_chunks = rows_per_w // CH
  n_groups = F // L

  mesh = plsc.VectorSubcoreMesh(core_axis_name="c", subcore_axis_name="s")

  @functools.partial(
      pl.kernel,
      mesh=mesh,
      compiler_params=pltpu.CompilerParams(needs_layout_passes=False),
      out_type=jax.ShapeDtypeStruct((R * F,), x.dtype),
      scratch_types=[
          pltpu.VMEM((F,), jnp.int32),
          [pltpu.VMEM((CH * F,), x.dtype) for _ in range(2)],
          [pltpu.VMEM((CH * F,), x.dtype) for _ in range(2)],
          [pltpu.SemaphoreType.DMA for _ in range(2)],
          [pltpu.SemaphoreType.DMA for _ in range(2)],
      ],
  )
  def k(x_hbm, perm_hbm, out_hbm, perm_v, in_v, out_v, in_sem, out_sem):
    wid = lax.axis_index("c") * NS + lax.axis_index("s")
    base = wid * rows_per_w
    pltpu.sync_copy(perm_hbm, perm_v)
    idxs = [perm_v[pl.ds(g * L, L)] for g in range(n_groups)]

    def in_copy(c, b):
      e0 = (base + c * CH) * F
      return pltpu.make_async_copy(
          x_hbm.at[pl.ds(e0, CH * F)], in_v[b], in_sem[b])

    def out_copy(c, b):
      e0 = (base + c * CH) * F
      return pltpu.make_async_copy(
          out_v[b], out_hbm.at[pl.ds(e0, CH * F)], out_sem[b])

    for b in range(2):
      in_copy(b, b).start()

    def chunk_pair(cc, carry):
      for b in range(2):
        c = cc * 2 + b
        in_copy(c, b).wait()

        @pl.when(c >= 2)
        def _():
          out_copy(c - 2, b).wait()

        @plsc.parallel_loop(0, CH, unroll=2)
        def row_body(i):
          off = i * F
          off_v = jnp.full((L,), off, dtype=jnp.int32)
          for g in range(n_groups):
            out_v[b][pl.ds(off + g * L, L)] = plsc.load_gather(
                in_v[b], [off_v + idxs[g]])

        out_copy(c, b).start()

        @pl.when(c + 2 < n_chunks)
        def _():
          in_copy(c + 2, b).start()

      return carry

    lax.fori_loop(0, n_chunks // 2, chunk_pair, 0)

    for b in range(2):
      out_copy(n_chunks - 2 + b, b).wait()

  return k(x.reshape(R * F), permutation).reshape(R, F)


def kernel(x, permutation):
  return _permute_columns_sc(x, permutation)


# speedup vs baseline: 3.3995x; 2.7291x over previous
import functools

import jax
import jax.numpy as jnp
from jax import lax
from jax.experimental import pallas as pl
from jax.experimental.pallas import tpu as pltpu
from jax.experimental.pallas import tpu_sc as plsc


def _permute_columns_sc(x, permutation):
  R, F = x.shape
  NC, NS, L = 2, 16, 16
  NW = NC * NS
  rows_per_w = R // NW
  CH = 64
  n_chunks = rows_per_w // CH
  n_groups = F // L

  mesh = plsc.VectorSubcoreMesh(core_axis_name="c", subcore_axis_name="s")

  @functools.partial(
      pl.kernel,
      mesh=mesh,
      compiler_params=pltpu.CompilerParams(needs_layout_passes=False),
      out_type=jax.ShapeDtypeStruct((R, F), x.dtype),
      scratch_types=[
          pltpu.VMEM((F,), jnp.int32),
          [pltpu.VMEM((CH, F), x.dtype) for _ in range(2)],
          [pltpu.VMEM((CH, F), x.dtype) for _ in range(2)],
          [pltpu.SemaphoreType.DMA for _ in range(2)],
          [pltpu.SemaphoreType.DMA for _ in range(2)],
      ],
  )
  def k(x_hbm, perm_hbm, out_hbm, perm_v, in_v, out_v, in_sem, out_sem):
    wid = lax.axis_index("c") * NS + lax.axis_index("s")
    base = wid * rows_per_w
    pltpu.sync_copy(perm_hbm, perm_v)
    idxs = [perm_v[pl.ds(g * L, L)] for g in range(n_groups)]

    def in_copy(c, b):
      r0 = base + c * CH
      return pltpu.make_async_copy(
          x_hbm.at[pl.ds(r0, CH)], in_v[b], in_sem[b])

    def out_copy(c, b):
      r0 = base + c * CH
      return pltpu.make_async_copy(
          out_v[b], out_hbm.at[pl.ds(r0, CH)], out_sem[b])

    for b in range(2):
      in_copy(b, b).start()

    def chunk_pair(cc, carry):
      for b in range(2):
        c = cc * 2 + b
        in_copy(c, b).wait()

        @pl.when(c >= 2)
        def _():
          out_copy(c - 2, b).wait()

        @plsc.parallel_loop(0, CH, unroll=2)
        def row_body(i):
          row = jnp.full((L,), i, dtype=jnp.int32)
          for g in range(n_groups):
            out_v[b][i, pl.ds(g * L, L)] = plsc.load_gather(
                in_v[b], [row, idxs[g]])

        out_copy(c, b).start()

        @pl.when(c + 2 < n_chunks)
        def _():
          in_copy(c + 2, b).start()

      return carry

    lax.fori_loop(0, n_chunks // 2, chunk_pair, 0)

    for b in range(2):
      out_copy(n_chunks - 2 + b, b).wait()

  return k(x, permutation)


def kernel(x, permutation):
  return _permute_columns_sc(x, permutation)
